# baseline (device time: 18044 ns/iter reference)
import jax
import jax.numpy as jnp
from jax import lax
from jax.experimental import pallas as pl
from jax.experimental.pallas import tpu as pltpu

N_DEV = 4
B, SQ, SKV, DH = 2, 256, 256, 64
HQ_TOTAL = 16
HQ_LOC = HQ_TOTAL // N_DEV
D_MODEL = 512
BLK = 64
NH = B * HQ_LOC


def kernel(x, Wq, K_ext, V_ext, Wo):

    my = lax.axis_index("i")

    def pack(t):
        t = t.reshape(B, SKV, HQ_TOTAL * DH)
        t = lax.dynamic_slice_in_dim(t, my * HQ_LOC * DH, HQ_LOC * DH, axis=2)
        return t.astype(jnp.bfloat16)

    k_loc = pack(K_ext)
    v_loc = pack(V_ext)

    def body(x_ref, wq_ref, k_ref, v_ref, wo_ref, out_ref,
             comm_ref, send_ref, send_sems, recv_sems):
        my_pos = lax.axis_index("i")
        p1 = my_pos ^ 1
        p2 = (N_DEV - 1) - my_pos

        barrier_sem = pltpu.get_barrier_semaphore()
        for nbr in (p1, p2):
            pl.semaphore_signal(
                barrier_sem, inc=1,
                device_id=(nbr,), device_id_type=pl.DeviceIdType.MESH,
            )
        pl.semaphore_wait(barrier_sem, 2)

        def exchange(slot, partner):
            return pltpu.make_async_remote_copy(
                src_ref=send_ref.at[slot],
                dst_ref=comm_ref.at[slot],
                send_sem=send_sems.at[slot],
                recv_sem=recv_sems.at[slot],
                device_id=(partner,),
                device_id_type=pl.DeviceIdType.MESH,
            )

        x2 = x_ref[...].reshape(B * SQ, D_MODEL).astype(jnp.bfloat16)
        wq = wq_ref[...].astype(jnp.bfloat16)
        wo = wo_ref[...].astype(jnp.bfloat16)

        GR = SQ // 2
        qb = lax.broadcasted_iota(jnp.int32, (SQ, SKV), 0) // BLK
        kb = lax.broadcasted_iota(jnp.int32, (SQ, SKV), 1) // BLK
        mask = kb <= qb

        def group_partial(g):
            b, half = g // 2, g % 2
            rows = slice(g * GR, (g + 1) * GR)
            qg = jnp.dot(x2[rows], wq,
                         preferred_element_type=jnp.float32)
            mask_g = mask[half * GR:(half + 1) * GR]
            parts = []
            for h in range(HQ_LOC):
                qbh = qg[:, h * DH:(h + 1) * DH].astype(jnp.bfloat16)
                kbh = k_ref[b][:, h * DH:(h + 1) * DH]
                s = lax.dot_general(
                    qbh, kbh, (((1,), (1,)), ((), ())),
                    preferred_element_type=jnp.float32,
                ) * 0.125
                w = jnp.exp(jnp.where(mask_g, s, -1e9))
                rcp = 1.0 / jnp.sum(w, axis=1, keepdims=True)
                ctx = jnp.dot(w.astype(jnp.bfloat16),
                              v_ref[b][:, h * DH:(h + 1) * DH],
                              preferred_element_type=jnp.float32)
                parts.append((ctx * rcp).astype(jnp.bfloat16))
            ctx_g = jnp.concatenate(parts, axis=1)
            return jnp.dot(ctx_g, wo,
                           preferred_element_type=jnp.float32)

        def s1_partner(g):
            return p1 if g % 2 == 0 else p2

        def s2_partner(g):
            return p2 if g % 2 == 0 else p1

        partials = [None] * 4
        accs = [None] * 4
        s1 = [None] * 4
        s2 = [None] * 4

        def start_stage1(g):
            partials[g] = group_partial(g)
            send_ref[2 * g] = partials[g].astype(jnp.bfloat16)
            s1[g] = exchange(2 * g, s1_partner(g))
            s1[g].start()

        def finish_stage1_start_stage2(g):
            s1[g].wait()
            accs[g] = partials[g] + comm_ref[2 * g].astype(jnp.float32)
            send_ref[2 * g + 1] = accs[g].astype(jnp.bfloat16)
            s2[g] = exchange(2 * g + 1, s2_partner(g))
            s2[g].start()

        start_stage1(0)
        start_stage1(1)
        finish_stage1_start_stage2(0)
        start_stage1(2)
        finish_stage1_start_stage2(1)
        start_stage1(3)
        finish_stage1_start_stage2(2)
        finish_stage1_start_stage2(3)

        for g in range(4):
            s2[g].wait()
            b, half = g // 2, g % 2
            out_ref[b, half * GR:(half + 1) * GR, :] = (
                accs[g] + comm_ref[2 * g + 1].astype(jnp.float32))

    return pl.pallas_call(
        body,
        out_shape=jax.ShapeDtypeStruct((B, SQ, D_MODEL), jnp.float32),
        in_specs=[pl.BlockSpec(memory_space=pltpu.VMEM)] * 5,
        out_specs=pl.BlockSpec(memory_space=pltpu.VMEM),
        scratch_shapes=[
            pltpu.VMEM((8, SQ // 2, D_MODEL), jnp.bfloat16),
            pltpu.VMEM((8, SQ // 2, D_MODEL), jnp.bfloat16),
            pltpu.SemaphoreType.DMA((8,)),
            pltpu.SemaphoreType.DMA((8,)),
        ],
        compiler_params=pltpu.CompilerParams(collective_id=0),
    )(x, Wq, k_loc, v_loc, Wo)


# device time: 18026 ns/iter; 1.0010x vs baseline; 1.0010x over previous
import jax
import jax.numpy as jnp
from jax import lax
from jax.experimental import pallas as pl
from jax.experimental.pallas import tpu as pltpu

N_DEV = 4
B, SQ, SKV, DH = 2, 256, 256, 64
HQ_TOTAL = 16
HQ_LOC = HQ_TOTAL // N_DEV
D_MODEL = 512
BLK = 64
NH = B * HQ_LOC


def kernel(x, Wq, K_ext, V_ext, Wo):

    my = lax.axis_index("i")

    def pack(t):
        t = t.reshape(B, SKV, HQ_TOTAL * DH)
        t = lax.dynamic_slice_in_dim(t, my * HQ_LOC * DH, HQ_LOC * DH, axis=2)
        return t.astype(jnp.bfloat16)

    k_loc = pack(K_ext)
    v_loc = pack(V_ext)

    def body(x_ref, wq_ref, k_ref, v_ref, wo_ref, out_ref,
             comm_ref, send_ref, send_sems, recv_sems):
        my_pos = lax.axis_index("i")
        p1 = my_pos ^ 1
        p2 = (N_DEV - 1) - my_pos

        barrier_sem = pltpu.get_barrier_semaphore()
        for nbr in (p1, p2):
            pl.semaphore_signal(
                barrier_sem, inc=1,
                device_id=(nbr,), device_id_type=pl.DeviceIdType.MESH,
            )
        pl.semaphore_wait(barrier_sem, 2)

        def exchange(slot, partner):
            return pltpu.make_async_remote_copy(
                src_ref=send_ref.at[slot],
                dst_ref=comm_ref.at[slot],
                send_sem=send_sems.at[slot],
                recv_sem=recv_sems.at[slot],
                device_id=(partner,),
                device_id_type=pl.DeviceIdType.MESH,
            )

        x2 = x_ref[...].reshape(B * SQ, D_MODEL).astype(jnp.bfloat16)
        wq = wq_ref[...].astype(jnp.bfloat16)
        wo = wo_ref[...].astype(jnp.bfloat16)

        GR = SQ // 2
        qb = lax.broadcasted_iota(jnp.int32, (SQ, SKV), 0) // BLK
        kb = lax.broadcasted_iota(jnp.int32, (SQ, SKV), 1) // BLK
        mask = kb <= qb

        q = jnp.dot(x2, wq, preferred_element_type=jnp.float32)

        def batch_partial(b):
            parts = []
            for h in range(HQ_LOC):
                qbh = q[b * SQ:(b + 1) * SQ, h * DH:(h + 1) * DH]
                qbh = qbh.astype(jnp.bfloat16)
                kbh = k_ref[b][:, h * DH:(h + 1) * DH]
                s = lax.dot_general(
                    qbh, kbh, (((1,), (1,)), ((), ())),
                    preferred_element_type=jnp.float32,
                ) * 0.125
                w = jnp.exp(jnp.where(mask, s, -1e9))
                rcp = 1.0 / jnp.sum(w, axis=1, keepdims=True)
                ctx = jnp.dot(w.astype(jnp.bfloat16),
                              v_ref[b][:, h * DH:(h + 1) * DH],
                              preferred_element_type=jnp.float32)
                parts.append((ctx * rcp).astype(jnp.bfloat16))
            ctx_b = jnp.concatenate(parts, axis=1)
            return jnp.dot(ctx_b, wo,
                           preferred_element_type=jnp.float32)

        def s1_partner(g):
            return p1 if g < 2 else p2

        def s2_partner(g):
            return p2 if g < 2 else p1

        partials = [None] * 4
        accs = [None] * 4
        s1 = [None] * 4
        s2 = [None] * 4

        def start_stage1(g, pg):
            partials[g] = pg
            send_ref[2 * g] = pg.astype(jnp.bfloat16)
            s1[g] = exchange(2 * g, s1_partner(g))
            s1[g].start()

        def finish_stage1_start_stage2(g):
            s1[g].wait()
            accs[g] = partials[g] + comm_ref[2 * g].astype(jnp.float32)
            send_ref[2 * g + 1] = accs[g].astype(jnp.bfloat16)
            s2[g] = exchange(2 * g + 1, s2_partner(g))
            s2[g].start()

        pA = batch_partial(0)
        start_stage1(0, pA[:GR])
        start_stage1(1, pA[GR:])
        pB = batch_partial(1)
        start_stage1(2, pB[:GR])
        finish_stage1_start_stage2(0)
        start_stage1(3, pB[GR:])
        finish_stage1_start_stage2(1)
        finish_stage1_start_stage2(2)
        finish_stage1_start_stage2(3)

        for g in range(4):
            s2[g].wait()
            b, half = g // 2, g % 2
            out_ref[b, half * GR:(half + 1) * GR, :] = (
                accs[g] + comm_ref[2 * g + 1].astype(jnp.float32))

    return pl.pallas_call(
        body,
        out_shape=jax.ShapeDtypeStruct((B, SQ, D_MODEL), jnp.float32),
        in_specs=[pl.BlockSpec(memory_space=pltpu.VMEM)] * 5,
        out_specs=pl.BlockSpec(memory_space=pltpu.VMEM),
        scratch_shapes=[
            pltpu.VMEM((8, SQ // 2, D_MODEL), jnp.bfloat16),
            pltpu.VMEM((8, SQ // 2, D_MODEL), jnp.bfloat16),
            pltpu.SemaphoreType.DMA((8,)),
            pltpu.SemaphoreType.DMA((8,)),
        ],
        compiler_params=pltpu.CompilerParams(collective_id=0),
    )(x, Wq, k_loc, v_loc, Wo)


# device time: 14574 ns/iter; 1.2381x vs baseline; 1.2369x over previous
import jax
import jax.numpy as jnp
from jax import lax
from jax.experimental import pallas as pl
from jax.experimental.pallas import tpu as pltpu

N_DEV = 4
B, SQ, SKV, DH = 2, 256, 256, 64
HQ_TOTAL = 16
HQ_LOC = HQ_TOTAL // N_DEV
D_MODEL = 512
BLK = 64
NH = B * HQ_LOC


def kernel(x, Wq, K_ext, V_ext, Wo):

    my = lax.axis_index("i")

    def pack(t):
        t = t.reshape(B, SKV, HQ_TOTAL * DH)
        t = lax.dynamic_slice_in_dim(t, my * HQ_LOC * DH, HQ_LOC * DH, axis=2)
        return t.astype(jnp.bfloat16)

    k_loc = pack(K_ext)
    v_loc = pack(V_ext)

    def body(x_ref, wq_ref, k_ref, v_ref, wo_ref, out_ref,
             comm_ref, send_ref, send_sems, recv_sems):
        my_pos = lax.axis_index("i")
        p1 = my_pos ^ 1
        p2 = (N_DEV - 1) - my_pos

        barrier_sem = pltpu.get_barrier_semaphore()
        for nbr in (p1, p2):
            pl.semaphore_signal(
                barrier_sem, inc=1,
                device_id=(nbr,), device_id_type=pl.DeviceIdType.MESH,
            )
        pl.semaphore_wait(barrier_sem, 2)

        def exchange(src, dst, sem_idx, partner):
            return pltpu.make_async_remote_copy(
                src_ref=src,
                dst_ref=dst,
                send_sem=send_sems.at[sem_idx],
                recv_sem=recv_sems.at[sem_idx],
                device_id=(partner,),
                device_id_type=pl.DeviceIdType.MESH,
            )

        x2 = x_ref[...].reshape(B * SQ, D_MODEL).astype(jnp.bfloat16)
        wq = (wq_ref[...] * 0.125).astype(jnp.bfloat16)
        wo = wo_ref[...].astype(jnp.bfloat16)

        GR = SQ // 2
        qb = lax.broadcasted_iota(jnp.int32, (SQ, SKV), 0) // BLK
        kb = lax.broadcasted_iota(jnp.int32, (SQ, SKV), 1) // BLK
        mask = kb <= qb

        q = jnp.dot(x2, wq, preferred_element_type=jnp.float32)

        def batch_partial(b):
            parts = []
            for h in range(HQ_LOC):
                qbh = q[b * SQ:(b + 1) * SQ, h * DH:(h + 1) * DH]
                qbh = qbh.astype(jnp.bfloat16)
                kbh = k_ref[b][:, h * DH:(h + 1) * DH]
                s = lax.dot_general(
                    qbh, kbh, (((1,), (1,)), ((), ())),
                    preferred_element_type=jnp.float32,
                )
                w = jnp.exp(jnp.where(mask, s, -1e9))
                rcp = 1.0 / jnp.sum(w, axis=1, keepdims=True)
                ctx = jnp.dot(w.astype(jnp.bfloat16),
                              v_ref[b][:, h * DH:(h + 1) * DH],
                              preferred_element_type=jnp.float32)
                parts.append((ctx * rcp).astype(jnp.bfloat16))
            ctx_b = jnp.concatenate(parts, axis=1)
            return jnp.dot(ctx_b, wo,
                           preferred_element_type=jnp.float32
                           ).astype(jnp.bfloat16)

        parts_ = [None] * 4
        accs = [None] * 4
        s1 = [None] * 4
        s2 = [None] * 4

        def start_stage1(c, pc):
            parts_[c] = pc
            send_ref[2 * c] = pc
            s1[c] = exchange(send_ref.at[2 * c], comm_ref.at[2 * c],
                             2 * c, p1 if c % 2 == 0 else p2)
            s1[c].start()

        def finish_stage1_start_stage2(c):
            s1[c].wait()
            accs[c] = parts_[c] + comm_ref[2 * c]
            send_ref[2 * c + 1] = accs[c]
            s2[c] = exchange(send_ref.at[2 * c + 1], comm_ref.at[2 * c + 1],
                             2 * c + 1, p2 if c % 2 == 0 else p1)
            s2[c].start()

        pA = batch_partial(0)
        start_stage1(0, pA[:GR])
        start_stage1(1, pA[GR:])

        pB = batch_partial(1)
        start_stage1(2, pB[:GR])
        start_stage1(3, pB[GR:])

        for c in range(4):
            finish_stage1_start_stage2(c)

        for c in range(4):
            s2[c].wait()
            b, half = c // 2, c % 2
            out_ref[b, half * GR:(half + 1) * GR, :] = (
                accs[c] + comm_ref[2 * c + 1])

    return pl.pallas_call(
        body,
        out_shape=jax.ShapeDtypeStruct((B, SQ, D_MODEL), jnp.bfloat16),
        in_specs=[pl.BlockSpec(memory_space=pltpu.VMEM)] * 5,
        out_specs=pl.BlockSpec(memory_space=pltpu.VMEM),
        scratch_shapes=[
            pltpu.VMEM((8, SQ // 2, D_MODEL), jnp.bfloat16),
            pltpu.VMEM((8, SQ // 2, D_MODEL), jnp.bfloat16),
            pltpu.SemaphoreType.DMA((8,)),
            pltpu.SemaphoreType.DMA((8,)),
        ],
        compiler_params=pltpu.CompilerParams(collective_id=0),
    )(x, Wq, k_loc, v_loc, Wo)
